# baseline (speedup 1.0000x reference)
import functools

import jax
import jax.numpy as jnp
from jax import lax
from jax.experimental import pallas as pl
from jax.experimental.pallas import tpu as pltpu
from jax.experimental.pallas import tpu_sc as plsc

B = 4096
N = 16384
L = 16
NC = 2
NS = 16
NW = NC * NS
ROWS_PER_W = B // NW
KU = 8
JBITS = 14
JMASK = (1 << JBITS) - 1


@functools.cache
def _build(nnzp: int):
    mesh = plsc.VectorSubcoreMesh(
        core_axis_name="c", subcore_axis_name="s", num_cores=NC, num_subcores=NS
    )

    @functools.partial(
        pl.kernel,
        out_type=jax.ShapeDtypeStruct((B, N), jnp.float32),
        mesh=mesh,
        compiler_params=pltpu.CompilerParams(needs_layout_passes=False),
        scratch_types=[
            pltpu.VMEM((nnzp,), jnp.int32),
            pltpu.VMEM((nnzp,), jnp.float32),
            pltpu.VMEM((N,), jnp.float32),
            pltpu.VMEM((N,), jnp.float32),
            pltpu.VMEM((N,), jnp.float32),
            pltpu.VMEM((N,), jnp.float32),
            pltpu.VMEM((N,), jnp.float32),
            pltpu.VMEM((N,), jnp.float32),
            pltpu.SemaphoreType.DMA,
            pltpu.SemaphoreType.DMA,
            pltpu.SemaphoreType.DMA,
            pltpu.SemaphoreType.DMA,
            pltpu.SemaphoreType.DMA,
            pltpu.SemaphoreType.DMA,
            pltpu.SemaphoreType.DMA,
            pltpu.SemaphoreType.DMA,
            pltpu.SemaphoreType.DMA,
            pltpu.SemaphoreType.DMA,
        ],
    )
    def sc_kernel(inputs_hbm, w3_hbm, b_hbm, vel_hbm, packed_hbm, out_hbm,
                  packed, vals, x0, x1, y0, y1, y2, y3,
                  sx0, sx1, sy0, sy1, sy2, sy3, sb0, sb1, sb2, sb3):
        wid = lax.axis_index("s") * NC + lax.axis_index("c")
        row0 = wid * ROWS_PER_W
        xs = (x0, x1)
        ys = (y0, y1, y2, y3)
        sxs = (sx0, sx1)
        sys_ = (sy0, sy1, sy2, sy3)
        sbs = (sb0, sb1, sb2, sb3)

        pltpu.sync_copy(packed_hbm, packed)
        pltpu.sync_copy(w3_hbm, y0.at[pl.ds(0, nnzp)])
        pltpu.sync_copy(vel_hbm, x0)

        @plsc.parallel_loop(0, nnzp // L, unroll=KU)
        def val_body(t):
            o = t * L
            pk = packed[pl.ds(o, L)]
            jv = lax.bitwise_and(pk, JMASK)
            g = plsc.load_gather(x0, [jv])
            vals[pl.ds(o, L)] = y0[pl.ds(o, L)] * g

        def k_loop(xbuf, ybuf):
            @plsc.parallel_loop(0, nnzp // L, unroll=KU)
            def k_body(t):
                o = t * L
                pk = packed[pl.ds(o, L)]
                jv = lax.bitwise_and(pk, JMASK)
                iv = lax.shift_right_logical(pk, JBITS)
                g = plsc.load_gather(xbuf, [jv])
                plsc.addupdate_scatter(ybuf, [iv], vals[pl.ds(o, L)] * g)

        pltpu.async_copy(inputs_hbm.at[row0], x0, sx0)
        pltpu.async_copy(inputs_hbm.at[row0 + 1], x1, sx1)
        for m in range(4):
            pltpu.async_copy(b_hbm, ys[m], sbs[m])

        row_end = row0 + ROWS_PER_W

        def quad_body(q, c):
            for m in range(4):
                rm = row0 + 4 * q + m
                xb, sx = xs[m % 2], sxs[m % 2]
                yb, sb = ys[m], sbs[m]
                mn = (m + 2) % 4
                ynxt, synxt, sbnxt = ys[mn], sys_[mn], sbs[mn]

                pltpu.make_async_copy(inputs_hbm.at[rm], xb, sx).wait()
                pltpu.make_async_copy(b_hbm, yb, sb).wait()
                k_loop(xb, yb)
                pltpu.async_copy(yb, out_hbm.at[rm], sys_[m])

                @pl.when(rm + 2 < row_end)
                def _():
                    pltpu.async_copy(inputs_hbm.at[rm + 2], xb, sx)

                if m < 2:
                    @pl.when(q > 0)
                    def _():
                        pltpu.make_async_copy(ynxt, out_hbm.at[rm - 2], synxt).wait()
                        pltpu.async_copy(b_hbm, ynxt, sbnxt)
                else:
                    @pl.when(rm + 2 < row_end)
                    def _():
                        pltpu.make_async_copy(ynxt, out_hbm.at[rm - 2], synxt).wait()
                        pltpu.async_copy(b_hbm, ynxt, sbnxt)
            return c

        lax.fori_loop(0, ROWS_PER_W // 4, quad_body, 0)

        for m in range(4):
            pltpu.make_async_copy(ys[m], out_hbm.at[row_end - 4 + m], sys_[m]).wait()

    return sc_kernel


def kernel(inputs, W3, b, velocity, I, J):
    nnz = W3.shape[0]
    chunk = L * KU
    nnzp = ((nnz + chunk - 1) // chunk) * chunk
    pad = nnzp - nnz
    packed = jnp.left_shift(I.astype(jnp.int32), JBITS) | J.astype(jnp.int32)
    packed = jnp.concatenate([packed, jnp.zeros((pad,), jnp.int32)])
    W3p = jnp.concatenate([W3, jnp.zeros((pad,), jnp.float32)])
    return _build(nnzp)(inputs, W3p, b, velocity, packed)
